# baseline (speedup 1.0000x reference)
import functools

import jax
import jax.numpy as jnp
from jax import lax
from jax.experimental import pallas as pl
from jax.experimental.pallas import tpu as pltpu
from jax.experimental.pallas import tpu_sc as plsc

B = 16384
T = 26
DIM = 32
NUM_WORKERS = 32
IBLK = 128
BLOCKS_PER_WORKER = B // (NUM_WORKERS * IBLK)

_mesh = plsc.VectorSubcoreMesh(core_axis_name="c", subcore_axis_name="s")


@functools.partial(
    pl.kernel,
    out_type=jax.ShapeDtypeStruct((T, 4, B // IBLK, 8, 128), jnp.float32),
    mesh=_mesh,
    scratch_types=[
        pltpu.VMEM((IBLK, T), jnp.int32),
        pltpu.VMEM((IBLK,), jnp.int32),
        pltpu.VMEM((IBLK,), jnp.int32),
        pltpu.VMEM((IBLK, DIM), jnp.float32),
        pltpu.VMEM((IBLK, DIM), jnp.float32),
        pltpu.VMEM((4, 8, 128), jnp.float32),
        pltpu.VMEM((4, 8, 128), jnp.float32),
        pltpu.SemaphoreType.DMA,
        pltpu.SemaphoreType.DMA,
        pltpu.SemaphoreType.DMA,
        pltpu.SemaphoreType.DMA,
    ],
    compiler_params=pltpu.CompilerParams(
        use_tc_tiling_on_sc=False, needs_layout_passes=False
    ),
)
def _embed_sc(
    idx_hbm,
    tbl_hbm,
    out_hbm,
    idx_blk,
    ib0,
    ib1,
    rows0,
    rows1,
    ov0,
    ov1,
    g0,
    g1,
    w0,
    w1,
):
    wid = lax.axis_index("s") * 2 + lax.axis_index("c")
    iota = lax.iota(jnp.int32, 16)

    def build_ib(jj, ib):
        col = jnp.full((16,), 0, jnp.int32) + jj
        for lc in range(IBLK // 16):
            vals = plsc.load_gather(idx_blk, [iota + (lc * 16), col])
            ib[pl.ds(lc * 16, 16)] = vals

    def transpose_to(rows, ov):
        for tr in range(4):
            for s in range(8):
                col = jnp.full((16,), 8 * tr + s, jnp.int32)
                for lc in range(IBLK // 16):
                    vals = plsc.load_gather(rows, [iota + (lc * 16), col])
                    ov[tr, s, pl.ds(lc * 16, 16)] = vals

    def gather_start(ib, rows, sem):
        return pltpu.async_copy(tbl_hbm.at[ib], rows, sem)

    def do_block(bi, carry):
        blk = wid * BLOCKS_PER_WORKER + bi
        pltpu.sync_copy(idx_hbm.at[pl.ds(blk * IBLK, IBLK)], idx_blk)
        build_ib(0, ib0)
        gather_start(ib0, rows0, g0)

        def pair(j2, carry):
            je = 2 * j2
            jo = je + 1
            build_ib(jo, ib1)
            gather_start(ib1, rows1, g1)
            pltpu.make_async_copy(tbl_hbm.at[ib0], rows0, g0).wait()

            @pl.when(jnp.logical_or(j2 > 0, bi > 0))
            def _():
                pltpu.make_async_copy(ov0, out_hbm.at[0, :, 0], w0).wait()

            transpose_to(rows0, ov0)
            pltpu.async_copy(ov0, out_hbm.at[je, :, blk], w0)
            @pl.when(j2 < (T // 2) - 1)
            def _():
                build_ib(je + 2, ib0)
                gather_start(ib0, rows0, g0)

            pltpu.make_async_copy(tbl_hbm.at[ib1], rows1, g1).wait()

            @pl.when(jnp.logical_or(j2 > 0, bi > 0))
            def _():
                pltpu.make_async_copy(ov1, out_hbm.at[0, :, 0], w1).wait()

            transpose_to(rows1, ov1)
            pltpu.async_copy(ov1, out_hbm.at[jo, :, blk], w1)
            return carry

        lax.fori_loop(0, T // 2, pair, 0)
        return carry

    lax.fori_loop(0, BLOCKS_PER_WORKER, do_block, 0)
    pltpu.make_async_copy(ov0, out_hbm.at[0, :, 0], w0).wait()
    pltpu.make_async_copy(ov1, out_hbm.at[0, :, 0], w1).wait()


def kernel(idx, weight):
    idx2d = idx.astype(jnp.int32)
    out5 = _embed_sc(idx2d, weight)
    return out5.transpose(2, 4, 0, 1, 3).reshape(B, T, DIM)


# speedup vs baseline: 1.1314x; 1.1314x over previous
import functools

import jax
import jax.numpy as jnp
from jax import lax
from jax.experimental import pallas as pl
from jax.experimental.pallas import tpu as pltpu
from jax.experimental.pallas import tpu_sc as plsc

B = 16384
T = 26
DIM = 32
NUM_WORKERS = 32
IBLK = 128
BLOCKS_PER_WORKER = B // (NUM_WORKERS * IBLK)

_mesh = plsc.VectorSubcoreMesh(core_axis_name="c", subcore_axis_name="s")


@functools.partial(
    pl.kernel,
    out_type=jax.ShapeDtypeStruct((T, 4, B // IBLK, 1024), jnp.float32),
    mesh=_mesh,
    scratch_types=[
        pltpu.VMEM((IBLK, T), jnp.int32),
        pltpu.VMEM((IBLK,), jnp.int32),
        pltpu.VMEM((IBLK,), jnp.int32),
        pltpu.VMEM((IBLK, DIM), jnp.float32),
        pltpu.VMEM((IBLK, DIM), jnp.float32),
        pltpu.VMEM((4, 1024), jnp.float32),
        pltpu.VMEM((4, 1024), jnp.float32),
        pltpu.SemaphoreType.DMA,
        pltpu.SemaphoreType.DMA,
        pltpu.SemaphoreType.DMA,
        pltpu.SemaphoreType.DMA,
    ],
    compiler_params=pltpu.CompilerParams(
        use_tc_tiling_on_sc=False, needs_layout_passes=False
    ),
)
def _embed_sc(
    idx_hbm,
    tbl_hbm,
    out_hbm,
    idx_blk,
    ib0,
    ib1,
    rows0,
    rows1,
    ov0,
    ov1,
    g0,
    g1,
    w0,
    w1,
):
    wid = lax.axis_index("s") * 2 + lax.axis_index("c")
    iota = lax.iota(jnp.int32, 16)

    def build_ib(jj, ib):
        col = jnp.full((16,), 0, jnp.int32) + jj
        vals = [
            plsc.load_gather(idx_blk, [iota + (lc * 16), col])
            for lc in range(IBLK // 16)
        ]
        for lc in range(IBLK // 16):
            ib[pl.ds(lc * 16, 16)] = vals[lc]

    def transpose_to(rows, ov):
        for tr in range(4):
            for s in range(8):
                col = jnp.full((16,), 8 * tr + s, jnp.int32)
                vals = [
                    plsc.load_gather(rows, [iota + (lc * 16), col])
                    for lc in range(8)
                ]
                for lc in range(8):
                    ov[tr, pl.ds(s * 128 + lc * 16, 16)] = vals[lc]

    def gather_start(ib, rows, sem):
        return pltpu.async_copy(tbl_hbm.at[ib], rows, sem)

    def do_block(bi, carry):
        blk = wid * BLOCKS_PER_WORKER + bi
        pltpu.sync_copy(idx_hbm.at[pl.ds(blk * IBLK, IBLK)], idx_blk)
        build_ib(0, ib0)
        gather_start(ib0, rows0, g0)

        def pair(j2, carry):
            je = 2 * j2
            jo = je + 1
            build_ib(jo, ib1)
            gather_start(ib1, rows1, g1)
            pltpu.make_async_copy(tbl_hbm.at[ib0], rows0, g0).wait()

            @pl.when(jnp.logical_or(j2 > 0, bi > 0))
            def _():
                pltpu.make_async_copy(ov0, out_hbm.at[0, :, 0], w0).wait()

            transpose_to(rows0, ov0)
            pltpu.async_copy(ov0, out_hbm.at[je, :, blk], w0)
            @pl.when(j2 < (T // 2) - 1)
            def _():
                build_ib(je + 2, ib0)
                gather_start(ib0, rows0, g0)

            pltpu.make_async_copy(tbl_hbm.at[ib1], rows1, g1).wait()

            @pl.when(jnp.logical_or(j2 > 0, bi > 0))
            def _():
                pltpu.make_async_copy(ov1, out_hbm.at[0, :, 0], w1).wait()

            transpose_to(rows1, ov1)
            pltpu.async_copy(ov1, out_hbm.at[jo, :, blk], w1)
            return carry

        lax.fori_loop(0, T // 2, pair, 0)
        return carry

    lax.fori_loop(0, BLOCKS_PER_WORKER, do_block, 0)
    pltpu.make_async_copy(ov0, out_hbm.at[0, :, 0], w0).wait()
    pltpu.make_async_copy(ov1, out_hbm.at[0, :, 0], w1).wait()


def kernel(idx, weight):
    idx2d = idx.astype(jnp.int32)
    out5 = _embed_sc(idx2d, weight)
    out5 = out5.reshape(T, 4, B // IBLK, 8, 128)
    return out5.transpose(2, 4, 0, 1, 3).reshape(B, T, DIM)
